# Initial kernel scaffold; baseline (speedup 1.0000x reference)
import jax
import jax.numpy as jnp
from jax.experimental import pallas as pl


def kernel(x, edge_index, W1, b1, W2, b2):
    raise NotImplementedError("write your pallas kernel here")


# speedup vs baseline: 33.0139x; 33.0139x over previous
import functools

import jax
import jax.numpy as jnp
from jax import lax
from jax.experimental import pallas as pl
from jax.experimental.pallas import tpu as pltpu
from jax.experimental.pallas import tpu_sc as plsc

N, E, D1, DH, D2 = 10000, 320000, 128, 512, 40
NPAD = 10240
D2P = 48
NW = 32
CH = 128
NCH = 80
EPAD = NW * NCH * CH
NBUF = 4
RPS = NPAD // 16
RB = 256


def _sc_mesh():
    return plsc.VectorSubcoreMesh(core_axis_name="c", subcore_axis_name="s")


def _make_deg_kernel():
    @functools.partial(
        pl.kernel,
        out_type=jax.ShapeDtypeStruct((2, NPAD, 16), jnp.float32),
        mesh=_sc_mesh(),
        compiler_params=pltpu.CompilerParams(use_tc_tiling_on_sc=False),
        scratch_types=[
            pltpu.VMEM((NCH, CH), jnp.int32),
            pltpu.VMEM((CH, 16), jnp.float32),
            pltpu.VMEM_SHARED((NPAD, 16), jnp.float32),
        ],
    )
    def deg_kernel(dst_hbm, ones_hbm, zeros_hbm, out_hbm, dst_v, ones_v, acc):
        c = lax.axis_index("c")
        s = lax.axis_index("s")
        wid = c * 16 + s
        pltpu.sync_copy(dst_hbm.at[wid], dst_v)
        pltpu.sync_copy(ones_hbm, ones_v)
        pltpu.sync_copy(zeros_hbm, acc.at[pl.ds(s * RPS, RPS)])
        plsc.subcore_barrier()

        def body(j, carry):
            pltpu.sync_copy(ones_v, acc.at[dst_v.at[j]], add=True)
            return carry

        lax.fori_loop(0, NCH, body, 0)
        plsc.subcore_barrier()
        pltpu.sync_copy(acc.at[pl.ds(s * RPS, RPS)],
                        out_hbm.at[c].at[pl.ds(s * RPS, RPS)])

    return deg_kernel


def _make_agg_split_kernel():
    DS = D1 // 2

    @functools.partial(
        pl.kernel,
        out_type=jax.ShapeDtypeStruct((2, NPAD, DS), jnp.float32),
        mesh=_sc_mesh(),
        compiler_params=pltpu.CompilerParams(use_tc_tiling_on_sc=False),
        scratch_types=[
            pltpu.VMEM((2, NCH, CH), jnp.int32),
            pltpu.VMEM((2, NCH, CH), jnp.int32),
            pltpu.VMEM((NBUF, CH, DS), jnp.float32),
            pltpu.VMEM_SHARED((NPAD, DS), jnp.float32),
            pltpu.SemaphoreType.DMA,
            pltpu.SemaphoreType.DMA,
            pltpu.SemaphoreType.DMA,
            pltpu.SemaphoreType.DMA,
        ],
    )
    def agg_kernel(src_hbm, dst_hbm, y_hbm, zeros_hbm, out_hbm,
                   src_v, dst_v, buf, acc, s0, s1, s2, s3):
        c = lax.axis_index("c")
        s = lax.axis_index("s")
        sems = [s0, s1, s2, s3]
        pltpu.sync_copy(src_hbm.at[pl.ds(2 * s, 2)], src_v)
        pltpu.sync_copy(dst_hbm.at[pl.ds(2 * s, 2)], dst_v)
        pltpu.sync_copy(zeros_hbm, acc.at[pl.ds(s * RPS, RPS)])
        plsc.subcore_barrier()

        def body(i, carry):
            j = i * NBUF
            for k in range(2):
                cps = []
                for b in range(NBUF):
                    cps.append(pltpu.async_copy(
                        y_hbm.at[c].at[src_v.at[k, j + b]], buf.at[b],
                        sems[b]))
                for b in range(NBUF):
                    cps[b].wait()
                    pltpu.sync_copy(buf.at[b], acc.at[dst_v.at[k, j + b]],
                                    add=True)
            return carry

        lax.fori_loop(0, NCH // NBUF, body, 0)
        plsc.subcore_barrier()
        pltpu.sync_copy(acc.at[pl.ds(s * RPS, RPS)],
                        out_hbm.at[c].at[pl.ds(s * RPS, RPS)])

    return agg_kernel


def _make_agg_kernel(D):
    @functools.partial(
        pl.kernel,
        out_type=jax.ShapeDtypeStruct((2, NPAD, D), jnp.float32),
        mesh=_sc_mesh(),
        compiler_params=pltpu.CompilerParams(use_tc_tiling_on_sc=False),
        scratch_types=[
            pltpu.VMEM((NCH, CH), jnp.int32),
            pltpu.VMEM((NCH, CH), jnp.int32),
            pltpu.VMEM((NBUF, CH, D), jnp.float32),
            pltpu.VMEM_SHARED((NPAD, D), jnp.float32),
            pltpu.SemaphoreType.DMA,
            pltpu.SemaphoreType.DMA,
            pltpu.SemaphoreType.DMA,
            pltpu.SemaphoreType.DMA,
        ],
    )
    def agg_kernel(src_hbm, dst_hbm, y_hbm, zeros_hbm, out_hbm,
                   src_v, dst_v, buf, acc, s0, s1, s2, s3):
        c = lax.axis_index("c")
        s = lax.axis_index("s")
        wid = c * 16 + s
        sems = [s0, s1, s2, s3]
        pltpu.sync_copy(src_hbm.at[wid], src_v)
        pltpu.sync_copy(dst_hbm.at[wid], dst_v)
        pltpu.sync_copy(zeros_hbm, acc.at[pl.ds(s * RPS, RPS)])
        plsc.subcore_barrier()

        def body(i, carry):
            j = i * NBUF
            cps = []
            for b in range(NBUF):
                cps.append(pltpu.async_copy(
                    y_hbm.at[src_v.at[j + b]], buf.at[b], sems[b]))
            for b in range(NBUF):
                cps[b].wait()
                pltpu.sync_copy(buf.at[b], acc.at[dst_v.at[j + b]], add=True)
            return carry

        lax.fori_loop(0, NCH // NBUF, body, 0)
        plsc.subcore_barrier()
        pltpu.sync_copy(acc.at[pl.ds(s * RPS, RPS)],
                        out_hbm.at[c].at[pl.ds(s * RPS, RPS)])

    return agg_kernel


def _tc_prep(degp, xp):
    def body(degp_ref, x_ref, y_ref, dinv_ref):
        d = degp_ref[0, :, 0:1] + degp_ref[1, :, 0:1] + 1.0
        dinv = lax.rsqrt(d)
        dinv_ref[...] = dinv
        y = x_ref[...] * dinv
        y_ref[0] = y[:, :D1 // 2]
        y_ref[1] = y[:, D1 // 2:]

    return pl.pallas_call(
        body,
        grid=(NPAD // RB,),
        in_specs=[
            pl.BlockSpec((2, RB, 16), lambda i: (0, i, 0)),
            pl.BlockSpec((RB, D1), lambda i: (i, 0)),
        ],
        out_specs=[
            pl.BlockSpec((2, RB, D1 // 2), lambda i: (0, i, 0)),
            pl.BlockSpec((RB, 1), lambda i: (i, 0)),
        ],
        out_shape=[
            jax.ShapeDtypeStruct((2, NPAD, D1 // 2), jnp.float32),
            jax.ShapeDtypeStruct((NPAD, 1), jnp.float32),
        ],
    )(degp, xp)


def _tc_mm(y, p, dinv, W1, b1r, W2p):
    def body(y_ref, p_ref, dinv_ref, w1_ref, b1_ref, w2_ref, y2_ref):
        dinv = dinv_ref[...]
        a = jnp.concatenate(
            [p_ref[0] + y_ref[0], p_ref[1] + y_ref[1]], axis=1) * dinv
        h = lax.dot_general(a, w1_ref[...], (((1,), (1,)), ((), ())),
                            preferred_element_type=jnp.float32)
        h = jnp.maximum(h + b1_ref[...], 0.0)
        m = lax.dot_general(h, w2_ref[...], (((1,), (1,)), ((), ())),
                            preferred_element_type=jnp.float32)
        y2_ref[...] = m * dinv

    return pl.pallas_call(
        body,
        grid=(NPAD // RB,),
        in_specs=[
            pl.BlockSpec((2, RB, D1 // 2), lambda i: (0, i, 0)),
            pl.BlockSpec((2, RB, D1 // 2), lambda i: (0, i, 0)),
            pl.BlockSpec((RB, 1), lambda i: (i, 0)),
            pl.BlockSpec((DH, D1), lambda i: (0, 0)),
            pl.BlockSpec((1, DH), lambda i: (0, 0)),
            pl.BlockSpec((D2P, DH), lambda i: (0, 0)),
        ],
        out_specs=pl.BlockSpec((RB, D2P), lambda i: (i, 0)),
        out_shape=jax.ShapeDtypeStruct((NPAD, D2P), jnp.float32),
    )(y, p, dinv, W1, b1r, W2p)


def _tc_out(q, y2, dinv, b2p):
    def body(q_ref, y2_ref, dinv_ref, b2_ref, o_ref):
        z = (q_ref[0] + q_ref[1] + y2_ref[...]) * dinv_ref[...] + b2_ref[...]
        col = lax.broadcasted_iota(jnp.int32, (RB, D2P), 1)
        zm = jnp.where(col < D2, z, -jnp.inf)
        mx = jnp.max(zm, axis=1, keepdims=True)
        lse = mx + jnp.log(jnp.sum(jnp.exp(zm - mx), axis=1, keepdims=True))
        o_ref[...] = z - lse

    return pl.pallas_call(
        body,
        grid=(NPAD // RB,),
        in_specs=[
            pl.BlockSpec((2, RB, D2P), lambda i: (0, i, 0)),
            pl.BlockSpec((RB, D2P), lambda i: (i, 0)),
            pl.BlockSpec((RB, 1), lambda i: (i, 0)),
            pl.BlockSpec((1, D2P), lambda i: (0, 0)),
        ],
        out_specs=pl.BlockSpec((RB, D2P), lambda i: (i, 0)),
        out_shape=jax.ShapeDtypeStruct((NPAD, D2P), jnp.float32),
    )(q, y2, dinv, b2p)


_deg_kernel = _make_deg_kernel()
_agg_d1 = _make_agg_split_kernel()
_agg_d2 = _make_agg_kernel(D2P)


def kernel(x, edge_index, W1, b1, W2, b2):
    f32 = jnp.float32
    npad_zone = NPAD - N
    pad_e = EPAD - E
    pad_idx = (N + (jnp.arange(pad_e, dtype=jnp.int32) % npad_zone))
    srcp = jnp.concatenate([edge_index[0], pad_idx]).reshape(NW, NCH, CH)
    dstp = jnp.concatenate([edge_index[1], pad_idx]).reshape(NW, NCH, CH)

    xp = jnp.concatenate([x, jnp.zeros((NPAD - N, D1), f32)], axis=0)
    ones_c = jnp.ones((CH, 16), f32)
    z1 = jnp.zeros((RPS, 16), f32)
    zd1 = jnp.zeros((RPS, D1 // 2), f32)
    zd2 = jnp.zeros((RPS, D2P), f32)
    W2p = jnp.concatenate([W2, jnp.zeros((D2P - D2, DH), f32)], axis=0)
    b1r = b1.reshape(1, DH)
    b2p = jnp.concatenate([b2, jnp.zeros((D2P - D2,), f32)]).reshape(1, D2P)

    degp = _deg_kernel(dstp, ones_c, z1)
    y, dinv = _tc_prep(degp, xp)
    p = _agg_d1(srcp, dstp, y, zd1)
    y2 = _tc_mm(y, p, dinv, W1, b1r, W2p)
    q = _agg_d2(srcp, dstp, y2, zd2)
    o = _tc_out(q, y2, dinv, b2p)
    return o[:N, :D2]


# speedup vs baseline: 35.5089x; 1.0756x over previous
import functools

import jax
import jax.numpy as jnp
from jax import lax
from jax.experimental import pallas as pl
from jax.experimental.pallas import tpu as pltpu
from jax.experimental.pallas import tpu_sc as plsc

N, E, D1, DH, D2 = 10000, 320000, 128, 512, 40
NPAD = 10240
D2P = 48
NW = 32
CH = 128
NCH = 80
EPAD = NW * NCH * CH
NBUF = 4
RPS = NPAD // 16
RB = 256


def _sc_mesh():
    return plsc.VectorSubcoreMesh(core_axis_name="c", subcore_axis_name="s")


def _make_deg_kernel():
    @functools.partial(
        pl.kernel,
        out_type=jax.ShapeDtypeStruct((2, NPAD, 16), jnp.float32),
        mesh=_sc_mesh(),
        compiler_params=pltpu.CompilerParams(use_tc_tiling_on_sc=False),
        scratch_types=[
            pltpu.VMEM((NCH, CH), jnp.int32),
            pltpu.VMEM((CH, 16), jnp.float32),
            pltpu.VMEM_SHARED((NPAD, 16), jnp.float32),
            pltpu.SemaphoreType.DMA,
        ],
    )
    def deg_kernel(dst_hbm, ones_hbm, zeros_hbm, out_hbm, dst_v, ones_v, acc,
                   sem):
        c = lax.axis_index("c")
        s = lax.axis_index("s")
        wid = c * 16 + s
        pltpu.sync_copy(dst_hbm.at[wid], dst_v)
        pltpu.sync_copy(ones_hbm, ones_v)
        pltpu.sync_copy(zeros_hbm, acc.at[pl.ds(s * RPS, RPS)])
        plsc.subcore_barrier()

        def fire(j, carry):
            pltpu.async_copy(ones_v, acc.at[dst_v.at[j]], sem, add=True)
            return carry

        lax.fori_loop(0, NCH, fire, 0)

        def drain(j, carry):
            pltpu.make_async_copy(ones_v, acc.at[dst_v.at[j]], sem).wait()
            return carry

        lax.fori_loop(0, NCH, drain, 0)
        plsc.subcore_barrier()
        pltpu.sync_copy(acc.at[pl.ds(s * RPS, RPS)],
                        out_hbm.at[c].at[pl.ds(s * RPS, RPS)])

    return deg_kernel


def _make_agg_split_kernel():
    DS = D1 // 2
    T = 2 * NCH

    @functools.partial(
        pl.kernel,
        out_type=jax.ShapeDtypeStruct((2, NPAD, DS), jnp.float32),
        mesh=_sc_mesh(),
        compiler_params=pltpu.CompilerParams(use_tc_tiling_on_sc=False),
        scratch_types=[
            pltpu.VMEM((T + NBUF, CH), jnp.int32),
            pltpu.VMEM((T, CH), jnp.int32),
            pltpu.VMEM((NBUF, CH, DS), jnp.float32),
            pltpu.VMEM_SHARED((NPAD, DS), jnp.float32),
            pltpu.SemaphoreType.DMA,
            pltpu.SemaphoreType.DMA,
            pltpu.SemaphoreType.DMA,
            pltpu.SemaphoreType.DMA,
            pltpu.SemaphoreType.DMA,
            pltpu.SemaphoreType.DMA,
            pltpu.SemaphoreType.DMA,
            pltpu.SemaphoreType.DMA,
        ],
    )
    def agg_kernel(src_hbm, dst_hbm, y_hbm, zeros_hbm, pad_hbm, out_hbm,
                   src_v, dst_v, buf, acc,
                   g0, g1, g2, g3, t0, t1, t2, t3):
        c = lax.axis_index("c")
        s = lax.axis_index("s")
        gsem = [g0, g1, g2, g3]
        ssem = [t0, t1, t2, t3]
        pltpu.sync_copy(src_hbm.at[2 * s], src_v.at[pl.ds(0, NCH)])
        pltpu.sync_copy(src_hbm.at[2 * s + 1], src_v.at[pl.ds(NCH, NCH)])
        pltpu.sync_copy(pad_hbm, src_v.at[pl.ds(T, NBUF)])
        pltpu.sync_copy(dst_hbm.at[2 * s], dst_v.at[pl.ds(0, NCH)])
        pltpu.sync_copy(dst_hbm.at[2 * s + 1], dst_v.at[pl.ds(NCH, NCH)])
        pltpu.sync_copy(zeros_hbm, acc.at[pl.ds(s * RPS, RPS)])
        plsc.subcore_barrier()

        for b in range(NBUF):
            pltpu.async_copy(y_hbm.at[c].at[src_v.at[b]], buf.at[b], gsem[b])

        def body(i, carry):
            j = i * NBUF
            for b in range(NBUF):
                pltpu.make_async_copy(
                    y_hbm.at[c].at[src_v.at[j + b]], buf.at[b],
                    gsem[b]).wait()
            scps = []
            for b in range(NBUF):
                scps.append(pltpu.async_copy(
                    buf.at[b], acc.at[dst_v.at[j + b]], ssem[b], add=True))
            for b in range(NBUF):
                scps[b].wait()
                pltpu.async_copy(y_hbm.at[c].at[src_v.at[j + NBUF + b]],
                                 buf.at[b], gsem[b])
            return carry

        lax.fori_loop(0, T // NBUF, body, 0)
        for b in range(NBUF):
            pltpu.make_async_copy(
                y_hbm.at[c].at[src_v.at[T + b]], buf.at[b], gsem[b]).wait()
        plsc.subcore_barrier()
        pltpu.sync_copy(acc.at[pl.ds(s * RPS, RPS)],
                        out_hbm.at[c].at[pl.ds(s * RPS, RPS)])

    return agg_kernel


def _make_agg_kernel(D):
    @functools.partial(
        pl.kernel,
        out_type=jax.ShapeDtypeStruct((2, NPAD, D), jnp.float32),
        mesh=_sc_mesh(),
        compiler_params=pltpu.CompilerParams(use_tc_tiling_on_sc=False),
        scratch_types=[
            pltpu.VMEM((NCH + NBUF, CH), jnp.int32),
            pltpu.VMEM((NCH, CH), jnp.int32),
            pltpu.VMEM((NBUF, CH, D), jnp.float32),
            pltpu.VMEM_SHARED((NPAD, D), jnp.float32),
            pltpu.SemaphoreType.DMA,
            pltpu.SemaphoreType.DMA,
            pltpu.SemaphoreType.DMA,
            pltpu.SemaphoreType.DMA,
            pltpu.SemaphoreType.DMA,
            pltpu.SemaphoreType.DMA,
            pltpu.SemaphoreType.DMA,
            pltpu.SemaphoreType.DMA,
        ],
    )
    def agg_kernel(src_hbm, dst_hbm, y_hbm, zeros_hbm, pad_hbm, out_hbm,
                   src_v, dst_v, buf, acc,
                   g0, g1, g2, g3, t0, t1, t2, t3):
        c = lax.axis_index("c")
        s = lax.axis_index("s")
        wid = c * 16 + s
        gsem = [g0, g1, g2, g3]
        ssem = [t0, t1, t2, t3]
        pltpu.sync_copy(src_hbm.at[wid], src_v.at[pl.ds(0, NCH)])
        pltpu.sync_copy(pad_hbm, src_v.at[pl.ds(NCH, NBUF)])
        pltpu.sync_copy(dst_hbm.at[wid], dst_v)
        pltpu.sync_copy(zeros_hbm, acc.at[pl.ds(s * RPS, RPS)])
        plsc.subcore_barrier()

        for b in range(NBUF):
            pltpu.async_copy(y_hbm.at[src_v.at[b]], buf.at[b], gsem[b])

        def body(i, carry):
            j = i * NBUF
            for b in range(NBUF):
                pltpu.make_async_copy(
                    y_hbm.at[src_v.at[j + b]], buf.at[b], gsem[b]).wait()
            scps = []
            for b in range(NBUF):
                scps.append(pltpu.async_copy(
                    buf.at[b], acc.at[dst_v.at[j + b]], ssem[b], add=True))
            for b in range(NBUF):
                scps[b].wait()
                pltpu.async_copy(y_hbm.at[src_v.at[j + NBUF + b]],
                                 buf.at[b], gsem[b])
            return carry

        lax.fori_loop(0, NCH // NBUF, body, 0)
        for b in range(NBUF):
            pltpu.make_async_copy(
                y_hbm.at[src_v.at[NCH + b]], buf.at[b], gsem[b]).wait()
        plsc.subcore_barrier()
        pltpu.sync_copy(acc.at[pl.ds(s * RPS, RPS)],
                        out_hbm.at[c].at[pl.ds(s * RPS, RPS)])

    return agg_kernel


def _tc_prep(degp, xp):
    def body(degp_ref, x_ref, y_ref, dinv_ref):
        d = degp_ref[0, :, 0:1] + degp_ref[1, :, 0:1] + 1.0
        dinv = lax.rsqrt(d)
        dinv_ref[...] = dinv
        y = x_ref[...] * dinv
        y_ref[0] = y[:, :D1 // 2]
        y_ref[1] = y[:, D1 // 2:]

    return pl.pallas_call(
        body,
        grid=(NPAD // RB,),
        in_specs=[
            pl.BlockSpec((2, RB, 16), lambda i: (0, i, 0)),
            pl.BlockSpec((RB, D1), lambda i: (i, 0)),
        ],
        out_specs=[
            pl.BlockSpec((2, RB, D1 // 2), lambda i: (0, i, 0)),
            pl.BlockSpec((RB, 1), lambda i: (i, 0)),
        ],
        out_shape=[
            jax.ShapeDtypeStruct((2, NPAD, D1 // 2), jnp.float32),
            jax.ShapeDtypeStruct((NPAD, 1), jnp.float32),
        ],
    )(degp, xp)


def _tc_mm(y, p, dinv, W1, b1r, W2p):
    def body(y_ref, p_ref, dinv_ref, w1_ref, b1_ref, w2_ref, y2_ref):
        dinv = dinv_ref[...]
        a = jnp.concatenate(
            [p_ref[0] + y_ref[0], p_ref[1] + y_ref[1]], axis=1) * dinv
        h = lax.dot_general(a, w1_ref[...], (((1,), (1,)), ((), ())),
                            preferred_element_type=jnp.float32)
        h = jnp.maximum(h + b1_ref[...], 0.0)
        m = lax.dot_general(h, w2_ref[...], (((1,), (1,)), ((), ())),
                            preferred_element_type=jnp.float32)
        y2_ref[...] = m * dinv

    return pl.pallas_call(
        body,
        grid=(NPAD // RB,),
        in_specs=[
            pl.BlockSpec((2, RB, D1 // 2), lambda i: (0, i, 0)),
            pl.BlockSpec((2, RB, D1 // 2), lambda i: (0, i, 0)),
            pl.BlockSpec((RB, 1), lambda i: (i, 0)),
            pl.BlockSpec((DH, D1), lambda i: (0, 0)),
            pl.BlockSpec((1, DH), lambda i: (0, 0)),
            pl.BlockSpec((D2P, DH), lambda i: (0, 0)),
        ],
        out_specs=pl.BlockSpec((RB, D2P), lambda i: (i, 0)),
        out_shape=jax.ShapeDtypeStruct((NPAD, D2P), jnp.float32),
    )(y, p, dinv, W1, b1r, W2p)


def _tc_out(q, y2, dinv, b2p):
    def body(q_ref, y2_ref, dinv_ref, b2_ref, o_ref):
        z = (q_ref[0] + q_ref[1] + y2_ref[...]) * dinv_ref[...] + b2_ref[...]
        col = lax.broadcasted_iota(jnp.int32, (RB, D2P), 1)
        zm = jnp.where(col < D2, z, -jnp.inf)
        mx = jnp.max(zm, axis=1, keepdims=True)
        lse = mx + jnp.log(jnp.sum(jnp.exp(zm - mx), axis=1, keepdims=True))
        o_ref[...] = z - lse

    return pl.pallas_call(
        body,
        grid=(NPAD // RB,),
        in_specs=[
            pl.BlockSpec((2, RB, D2P), lambda i: (0, i, 0)),
            pl.BlockSpec((RB, D2P), lambda i: (i, 0)),
            pl.BlockSpec((RB, 1), lambda i: (i, 0)),
            pl.BlockSpec((1, D2P), lambda i: (0, 0)),
        ],
        out_specs=pl.BlockSpec((RB, D2P), lambda i: (i, 0)),
        out_shape=jax.ShapeDtypeStruct((NPAD, D2P), jnp.float32),
    )(q, y2, dinv, b2p)


_deg_kernel = _make_deg_kernel()
_agg_d1 = _make_agg_split_kernel()
_agg_d2 = _make_agg_kernel(D2P)


def kernel(x, edge_index, W1, b1, W2, b2):
    f32 = jnp.float32
    npad_zone = NPAD - N
    pad_e = EPAD - E
    pad_idx = (N + (jnp.arange(pad_e, dtype=jnp.int32) % npad_zone))
    srcp = jnp.concatenate([edge_index[0], pad_idx]).reshape(NW, NCH, CH)
    dstp = jnp.concatenate([edge_index[1], pad_idx]).reshape(NW, NCH, CH)

    xp = jnp.concatenate([x, jnp.zeros((NPAD - N, D1), f32)], axis=0)
    ones_c = jnp.ones((CH, 16), f32)
    z1 = jnp.zeros((RPS, 16), f32)
    zd1 = jnp.zeros((RPS, D1 // 2), f32)
    zd2 = jnp.zeros((RPS, D2P), f32)
    W2p = jnp.concatenate([W2, jnp.zeros((D2P - D2, DH), f32)], axis=0)
    b1r = b1.reshape(1, DH)
    b2p = jnp.concatenate([b2, jnp.zeros((D2P - D2,), f32)]).reshape(1, D2P)

    padc = (N + (jnp.arange(NBUF * CH, dtype=jnp.int32)
                 % npad_zone)).reshape(NBUF, CH)

    degp = _deg_kernel(dstp, ones_c, z1)
    y, dinv = _tc_prep(degp, xp)
    p = _agg_d1(srcp, dstp, y, zd1, padc)
    y2 = _tc_mm(y, p, dinv, W1, b1r, W2p)
    q = _agg_d2(srcp, dstp, y2, zd2, padc)
    o = _tc_out(q, y2, dinv, b2p)
    return o[:N, :D2]


# speedup vs baseline: 40.8846x; 1.1514x over previous
import functools

import jax
import jax.numpy as jnp
from jax import lax
from jax.experimental import pallas as pl
from jax.experimental.pallas import tpu as pltpu
from jax.experimental.pallas import tpu_sc as plsc

N, E, D1, DH, D2 = 10000, 320000, 128, 512, 40
NPAD = 10240
D2P = 48
NW = 32
CH = 128
NCH = 80
EPAD = NW * NCH * CH
NBUF = 4
RPS = NPAD // 16
RB = 1024


def _sc_mesh():
    return plsc.VectorSubcoreMesh(core_axis_name="c", subcore_axis_name="s")


def _make_deg_kernel():
    @functools.partial(
        pl.kernel,
        out_type=jax.ShapeDtypeStruct((2, NPAD, 16), jnp.float32),
        mesh=_sc_mesh(),
        compiler_params=pltpu.CompilerParams(use_tc_tiling_on_sc=False),
        scratch_types=[
            pltpu.VMEM((NCH, CH), jnp.int32),
            pltpu.VMEM((CH, 16), jnp.float32),
            pltpu.VMEM_SHARED((NPAD, 16), jnp.float32),
            pltpu.SemaphoreType.DMA,
        ],
    )
    def deg_kernel(dst_hbm, ones_hbm, zeros_hbm, out_hbm, dst_v, ones_v, acc,
                   sem):
        c = lax.axis_index("c")
        s = lax.axis_index("s")
        wid = c * 16 + s
        pltpu.sync_copy(dst_hbm.at[wid], dst_v)
        pltpu.sync_copy(ones_hbm, ones_v)
        pltpu.sync_copy(zeros_hbm, acc.at[pl.ds(s * RPS, RPS)])
        plsc.subcore_barrier()

        def fire(j, carry):
            pltpu.async_copy(ones_v, acc.at[dst_v.at[j]], sem, add=True)
            return carry

        lax.fori_loop(0, NCH, fire, 0)

        def drain(j, carry):
            pltpu.make_async_copy(ones_v, acc.at[dst_v.at[j]], sem).wait()
            return carry

        lax.fori_loop(0, NCH, drain, 0)
        plsc.subcore_barrier()
        pltpu.sync_copy(acc.at[pl.ds(s * RPS, RPS)],
                        out_hbm.at[c].at[pl.ds(s * RPS, RPS)])

    return deg_kernel


def _make_agg_split_kernel():
    DS = D1 // 2
    T = 2 * NCH

    @functools.partial(
        pl.kernel,
        out_type=jax.ShapeDtypeStruct((2, NPAD, DS), jnp.float32),
        mesh=_sc_mesh(),
        compiler_params=pltpu.CompilerParams(use_tc_tiling_on_sc=False),
        scratch_types=[
            pltpu.VMEM((T + NBUF, CH), jnp.int32),
            pltpu.VMEM((T, CH), jnp.int32),
            pltpu.VMEM((NBUF, CH, DS), jnp.float32),
            pltpu.VMEM_SHARED((NPAD, DS), jnp.float32),
            pltpu.SemaphoreType.DMA,
            pltpu.SemaphoreType.DMA,
            pltpu.SemaphoreType.DMA,
            pltpu.SemaphoreType.DMA,
            pltpu.SemaphoreType.DMA,
            pltpu.SemaphoreType.DMA,
            pltpu.SemaphoreType.DMA,
            pltpu.SemaphoreType.DMA,
        ],
    )
    def agg_kernel(src_hbm, dst_hbm, y_hbm, zeros_hbm, pad_hbm, out_hbm,
                   src_v, dst_v, buf, acc,
                   g0, g1, g2, g3, t0, t1, t2, t3):
        c = lax.axis_index("c")
        s = lax.axis_index("s")
        gsem = [g0, g1, g2, g3]
        ssem = [t0, t1, t2, t3]
        pltpu.sync_copy(src_hbm.at[2 * s], src_v.at[pl.ds(0, NCH)])
        pltpu.sync_copy(src_hbm.at[2 * s + 1], src_v.at[pl.ds(NCH, NCH)])
        pltpu.sync_copy(pad_hbm, src_v.at[pl.ds(T, NBUF)])
        pltpu.sync_copy(dst_hbm.at[2 * s], dst_v.at[pl.ds(0, NCH)])
        pltpu.sync_copy(dst_hbm.at[2 * s + 1], dst_v.at[pl.ds(NCH, NCH)])
        pltpu.sync_copy(zeros_hbm, acc.at[pl.ds(s * RPS, RPS)])
        plsc.subcore_barrier()

        for b in range(NBUF):
            pltpu.async_copy(y_hbm.at[c].at[src_v.at[b]], buf.at[b], gsem[b])

        def body(i, carry):
            j = i * NBUF
            for b in range(NBUF):
                pltpu.make_async_copy(
                    y_hbm.at[c].at[src_v.at[j + b]], buf.at[b],
                    gsem[b]).wait()
            scps = []
            for b in range(NBUF):
                scps.append(pltpu.async_copy(
                    buf.at[b], acc.at[dst_v.at[j + b]], ssem[b], add=True))
            for b in range(NBUF):
                scps[b].wait()
                pltpu.async_copy(y_hbm.at[c].at[src_v.at[j + NBUF + b]],
                                 buf.at[b], gsem[b])
            return carry

        lax.fori_loop(0, T // NBUF, body, 0)
        for b in range(NBUF):
            pltpu.make_async_copy(
                y_hbm.at[c].at[src_v.at[T + b]], buf.at[b], gsem[b]).wait()
        plsc.subcore_barrier()
        pltpu.sync_copy(acc.at[pl.ds(s * RPS, RPS)],
                        out_hbm.at[c].at[pl.ds(s * RPS, RPS)])

    return agg_kernel


def _make_agg_kernel(D):
    @functools.partial(
        pl.kernel,
        out_type=jax.ShapeDtypeStruct((2, NPAD, D), jnp.float32),
        mesh=_sc_mesh(),
        compiler_params=pltpu.CompilerParams(use_tc_tiling_on_sc=False),
        scratch_types=[
            pltpu.VMEM((NCH + NBUF, CH), jnp.int32),
            pltpu.VMEM((NCH, CH), jnp.int32),
            pltpu.VMEM((NBUF, CH, D), jnp.float32),
            pltpu.VMEM_SHARED((NPAD, D), jnp.float32),
            pltpu.SemaphoreType.DMA,
            pltpu.SemaphoreType.DMA,
            pltpu.SemaphoreType.DMA,
            pltpu.SemaphoreType.DMA,
            pltpu.SemaphoreType.DMA,
            pltpu.SemaphoreType.DMA,
            pltpu.SemaphoreType.DMA,
            pltpu.SemaphoreType.DMA,
        ],
    )
    def agg_kernel(src_hbm, dst_hbm, y_hbm, zeros_hbm, pad_hbm, out_hbm,
                   src_v, dst_v, buf, acc,
                   g0, g1, g2, g3, t0, t1, t2, t3):
        c = lax.axis_index("c")
        s = lax.axis_index("s")
        wid = c * 16 + s
        gsem = [g0, g1, g2, g3]
        ssem = [t0, t1, t2, t3]
        pltpu.sync_copy(src_hbm.at[wid], src_v.at[pl.ds(0, NCH)])
        pltpu.sync_copy(pad_hbm, src_v.at[pl.ds(NCH, NBUF)])
        pltpu.sync_copy(dst_hbm.at[wid], dst_v)
        pltpu.sync_copy(zeros_hbm, acc.at[pl.ds(s * RPS, RPS)])
        plsc.subcore_barrier()

        for b in range(NBUF):
            pltpu.async_copy(y_hbm.at[src_v.at[b]], buf.at[b], gsem[b])

        def body(i, carry):
            j = i * NBUF
            for b in range(NBUF):
                pltpu.make_async_copy(
                    y_hbm.at[src_v.at[j + b]], buf.at[b], gsem[b]).wait()
            scps = []
            for b in range(NBUF):
                scps.append(pltpu.async_copy(
                    buf.at[b], acc.at[dst_v.at[j + b]], ssem[b], add=True))
            for b in range(NBUF):
                scps[b].wait()
                pltpu.async_copy(y_hbm.at[src_v.at[j + NBUF + b]],
                                 buf.at[b], gsem[b])
            return carry

        lax.fori_loop(0, NCH // NBUF, body, 0)
        for b in range(NBUF):
            pltpu.make_async_copy(
                y_hbm.at[src_v.at[NCH + b]], buf.at[b], gsem[b]).wait()
        plsc.subcore_barrier()
        pltpu.sync_copy(acc.at[pl.ds(s * RPS, RPS)],
                        out_hbm.at[c].at[pl.ds(s * RPS, RPS)])

    return agg_kernel


def _tc_prep(degp, xp):
    def body(degp_ref, x_ref, y_ref, dinv_ref):
        d = degp_ref[0, :, 0:1] + degp_ref[1, :, 0:1] + 1.0
        dinv = lax.rsqrt(d)
        dinv_ref[...] = dinv
        y = x_ref[...] * dinv
        y_ref[0] = y[:, :D1 // 2]
        y_ref[1] = y[:, D1 // 2:]

    return pl.pallas_call(
        body,
        grid=(NPAD // RB,),
        in_specs=[
            pl.BlockSpec((2, RB, 16), lambda i: (0, i, 0)),
            pl.BlockSpec((RB, D1), lambda i: (i, 0)),
        ],
        out_specs=[
            pl.BlockSpec((2, RB, D1 // 2), lambda i: (0, i, 0)),
            pl.BlockSpec((RB, 1), lambda i: (i, 0)),
        ],
        out_shape=[
            jax.ShapeDtypeStruct((2, NPAD, D1 // 2), jnp.float32),
            jax.ShapeDtypeStruct((NPAD, 1), jnp.float32),
        ],
    )(degp, xp)


def _tc_mm(y, p, dinv, W1, b1r, W2p):
    def body(y_ref, p_ref, dinv_ref, w1_ref, b1_ref, w2_ref, y2_ref):
        dinv = dinv_ref[...]
        a = jnp.concatenate(
            [p_ref[0] + y_ref[0], p_ref[1] + y_ref[1]], axis=1) * dinv
        h = lax.dot_general(a, w1_ref[...], (((1,), (1,)), ((), ())),
                            preferred_element_type=jnp.float32)
        h = jnp.maximum(h + b1_ref[...], 0.0)
        m = lax.dot_general(h, w2_ref[...], (((1,), (1,)), ((), ())),
                            preferred_element_type=jnp.float32)
        y2_ref[...] = m * dinv

    return pl.pallas_call(
        body,
        grid=(NPAD // RB,),
        in_specs=[
            pl.BlockSpec((2, RB, D1 // 2), lambda i: (0, i, 0)),
            pl.BlockSpec((2, RB, D1 // 2), lambda i: (0, i, 0)),
            pl.BlockSpec((RB, 1), lambda i: (i, 0)),
            pl.BlockSpec((DH, D1), lambda i: (0, 0)),
            pl.BlockSpec((1, DH), lambda i: (0, 0)),
            pl.BlockSpec((D2P, DH), lambda i: (0, 0)),
        ],
        out_specs=pl.BlockSpec((RB, D2P), lambda i: (i, 0)),
        out_shape=jax.ShapeDtypeStruct((NPAD, D2P), jnp.float32),
    )(y, p, dinv, W1, b1r, W2p)


def _tc_out(q, y2, dinv, b2p):
    def body(q_ref, y2_ref, dinv_ref, b2_ref, o_ref):
        z = (q_ref[0] + q_ref[1] + y2_ref[...]) * dinv_ref[...] + b2_ref[...]
        col = lax.broadcasted_iota(jnp.int32, (RB, D2P), 1)
        zm = jnp.where(col < D2, z, -jnp.inf)
        mx = jnp.max(zm, axis=1, keepdims=True)
        lse = mx + jnp.log(jnp.sum(jnp.exp(zm - mx), axis=1, keepdims=True))
        o_ref[...] = z - lse

    return pl.pallas_call(
        body,
        grid=(NPAD // RB,),
        in_specs=[
            pl.BlockSpec((2, RB, D2P), lambda i: (0, i, 0)),
            pl.BlockSpec((RB, D2P), lambda i: (i, 0)),
            pl.BlockSpec((RB, 1), lambda i: (i, 0)),
            pl.BlockSpec((1, D2P), lambda i: (0, 0)),
        ],
        out_specs=pl.BlockSpec((RB, D2P), lambda i: (i, 0)),
        out_shape=jax.ShapeDtypeStruct((NPAD, D2P), jnp.float32),
    )(q, y2, dinv, b2p)


_deg_kernel = _make_deg_kernel()
_agg_d1 = _make_agg_split_kernel()
_agg_d2 = _make_agg_kernel(D2P)


def kernel(x, edge_index, W1, b1, W2, b2):
    f32 = jnp.float32
    npad_zone = NPAD - N
    pad_e = EPAD - E
    pad_idx = (N + (jnp.arange(pad_e, dtype=jnp.int32) % npad_zone))
    srcp = jnp.concatenate([edge_index[0], pad_idx]).reshape(NW, NCH, CH)
    dstp = jnp.concatenate([edge_index[1], pad_idx]).reshape(NW, NCH, CH)

    xp = jnp.concatenate([x, jnp.zeros((NPAD - N, D1), f32)], axis=0)
    ones_c = jnp.ones((CH, 16), f32)
    z1 = jnp.zeros((RPS, 16), f32)
    zd1 = jnp.zeros((RPS, D1 // 2), f32)
    zd2 = jnp.zeros((RPS, D2P), f32)
    W2p = jnp.concatenate([W2, jnp.zeros((D2P - D2, DH), f32)], axis=0)
    b1r = b1.reshape(1, DH)
    b2p = jnp.concatenate([b2, jnp.zeros((D2P - D2,), f32)]).reshape(1, D2P)

    padc = (N + (jnp.arange(NBUF * CH, dtype=jnp.int32)
                 % npad_zone)).reshape(NBUF, CH)

    degp = _deg_kernel(dstp, ones_c, z1)
    y, dinv = _tc_prep(degp, xp)
    p = _agg_d1(srcp, dstp, y, zd1, padc)
    y2 = _tc_mm(y, p, dinv, W1, b1r, W2p)
    q = _agg_d2(srcp, dstp, y2, zd2, padc)
    o = _tc_out(q, y2, dinv, b2p)
    return o[:N, :D2]


# speedup vs baseline: 41.7603x; 1.0214x over previous
import functools

import jax
import jax.numpy as jnp
from jax import lax
from jax.experimental import pallas as pl
from jax.experimental.pallas import tpu as pltpu
from jax.experimental.pallas import tpu_sc as plsc

N, E, D1, DH, D2 = 10000, 320000, 128, 512, 40
NPAD = 10240
D2P = 48
NW = 32
CH = 128
NCH = 80
EPAD = NW * NCH * CH
NBUF = 5
RPS = NPAD // 16
RB = 1024


def _sc_mesh():
    return plsc.VectorSubcoreMesh(core_axis_name="c", subcore_axis_name="s")


def _make_deg_kernel():
    @functools.partial(
        pl.kernel,
        out_type=jax.ShapeDtypeStruct((2, NPAD, 16), jnp.float32),
        mesh=_sc_mesh(),
        compiler_params=pltpu.CompilerParams(use_tc_tiling_on_sc=False),
        scratch_types=[
            pltpu.VMEM((NCH, CH), jnp.int32),
            pltpu.VMEM((CH, 16), jnp.float32),
            pltpu.VMEM_SHARED((NPAD, 16), jnp.float32),
            pltpu.SemaphoreType.DMA,
        ],
    )
    def deg_kernel(dst_hbm, ones_hbm, zeros_hbm, out_hbm, dst_v, ones_v, acc,
                   sem):
        c = lax.axis_index("c")
        s = lax.axis_index("s")
        wid = c * 16 + s
        pltpu.sync_copy(dst_hbm.at[wid], dst_v)
        pltpu.sync_copy(ones_hbm, ones_v)
        pltpu.sync_copy(zeros_hbm, acc.at[pl.ds(s * RPS, RPS)])
        plsc.subcore_barrier()

        def fire(j, carry):
            pltpu.async_copy(ones_v, acc.at[dst_v.at[j]], sem, add=True)
            return carry

        lax.fori_loop(0, NCH, fire, 0)

        def drain(j, carry):
            pltpu.make_async_copy(ones_v, acc.at[dst_v.at[j]], sem).wait()
            return carry

        lax.fori_loop(0, NCH, drain, 0)
        plsc.subcore_barrier()
        pltpu.sync_copy(acc.at[pl.ds(s * RPS, RPS)],
                        out_hbm.at[c].at[pl.ds(s * RPS, RPS)])

    return deg_kernel


def _make_agg_split_kernel():
    DS = D1 // 2
    T = 2 * NCH

    @functools.partial(
        pl.kernel,
        out_type=jax.ShapeDtypeStruct((2, NPAD, DS), jnp.float32),
        mesh=_sc_mesh(),
        compiler_params=pltpu.CompilerParams(use_tc_tiling_on_sc=False),
        scratch_types=[
            pltpu.VMEM((T + NBUF, CH), jnp.int32),
            pltpu.VMEM((T, CH), jnp.int32),
            pltpu.VMEM((NBUF, CH, DS), jnp.float32),
            pltpu.VMEM_SHARED((NPAD, DS), jnp.float32),
        ] + [pltpu.SemaphoreType.DMA] * (2 * NBUF),
    )
    def agg_kernel(src_hbm, dst_hbm, y_hbm, zeros_hbm, pad_hbm, out_hbm,
                   src_v, dst_v, buf, acc, *sems):
        c = lax.axis_index("c")
        s = lax.axis_index("s")
        gsem = sems[:NBUF]
        ssem = sems[NBUF:]
        pltpu.sync_copy(src_hbm.at[2 * s], src_v.at[pl.ds(0, NCH)])
        pltpu.sync_copy(src_hbm.at[2 * s + 1], src_v.at[pl.ds(NCH, NCH)])
        pltpu.sync_copy(pad_hbm, src_v.at[pl.ds(T, NBUF)])
        pltpu.sync_copy(dst_hbm.at[2 * s], dst_v.at[pl.ds(0, NCH)])
        pltpu.sync_copy(dst_hbm.at[2 * s + 1], dst_v.at[pl.ds(NCH, NCH)])
        pltpu.sync_copy(zeros_hbm, acc.at[pl.ds(s * RPS, RPS)])
        plsc.subcore_barrier()

        for b in range(NBUF):
            pltpu.async_copy(y_hbm.at[c].at[src_v.at[b]], buf.at[b], gsem[b])

        def body(i, carry):
            j = i * NBUF
            for b in range(NBUF):
                pltpu.make_async_copy(
                    y_hbm.at[c].at[src_v.at[j + b]], buf.at[b],
                    gsem[b]).wait()
            scps = []
            for b in range(NBUF):
                scps.append(pltpu.async_copy(
                    buf.at[b], acc.at[dst_v.at[j + b]], ssem[b], add=True))
            for b in range(NBUF):
                scps[b].wait()
                pltpu.async_copy(y_hbm.at[c].at[src_v.at[j + NBUF + b]],
                                 buf.at[b], gsem[b])
            return carry

        lax.fori_loop(0, T // NBUF, body, 0)
        for b in range(NBUF):
            pltpu.make_async_copy(
                y_hbm.at[c].at[src_v.at[T + b]], buf.at[b], gsem[b]).wait()
        plsc.subcore_barrier()
        pltpu.sync_copy(acc.at[pl.ds(s * RPS, RPS)],
                        out_hbm.at[c].at[pl.ds(s * RPS, RPS)])

    return agg_kernel


def _make_agg_kernel(D):
    @functools.partial(
        pl.kernel,
        out_type=jax.ShapeDtypeStruct((2, NPAD, D), jnp.float32),
        mesh=_sc_mesh(),
        compiler_params=pltpu.CompilerParams(use_tc_tiling_on_sc=False),
        scratch_types=[
            pltpu.VMEM((NCH + NBUF, CH), jnp.int32),
            pltpu.VMEM((NCH, CH), jnp.int32),
            pltpu.VMEM((NBUF, CH, D), jnp.float32),
            pltpu.VMEM_SHARED((NPAD, D), jnp.float32),
        ] + [pltpu.SemaphoreType.DMA] * (2 * NBUF),
    )
    def agg_kernel(src_hbm, dst_hbm, y_hbm, zeros_hbm, pad_hbm, out_hbm,
                   src_v, dst_v, buf, acc, *sems):
        c = lax.axis_index("c")
        s = lax.axis_index("s")
        wid = c * 16 + s
        gsem = sems[:NBUF]
        ssem = sems[NBUF:]
        pltpu.sync_copy(src_hbm.at[wid], src_v.at[pl.ds(0, NCH)])
        pltpu.sync_copy(pad_hbm, src_v.at[pl.ds(NCH, NBUF)])
        pltpu.sync_copy(dst_hbm.at[wid], dst_v)
        pltpu.sync_copy(zeros_hbm, acc.at[pl.ds(s * RPS, RPS)])
        plsc.subcore_barrier()

        for b in range(NBUF):
            pltpu.async_copy(y_hbm.at[src_v.at[b]], buf.at[b], gsem[b])

        def body(i, carry):
            j = i * NBUF
            for b in range(NBUF):
                pltpu.make_async_copy(
                    y_hbm.at[src_v.at[j + b]], buf.at[b], gsem[b]).wait()
            scps = []
            for b in range(NBUF):
                scps.append(pltpu.async_copy(
                    buf.at[b], acc.at[dst_v.at[j + b]], ssem[b], add=True))
            for b in range(NBUF):
                scps[b].wait()
                pltpu.async_copy(y_hbm.at[src_v.at[j + NBUF + b]],
                                 buf.at[b], gsem[b])
            return carry

        lax.fori_loop(0, NCH // NBUF, body, 0)
        for b in range(NBUF):
            pltpu.make_async_copy(
                y_hbm.at[src_v.at[NCH + b]], buf.at[b], gsem[b]).wait()
        plsc.subcore_barrier()
        pltpu.sync_copy(acc.at[pl.ds(s * RPS, RPS)],
                        out_hbm.at[c].at[pl.ds(s * RPS, RPS)])

    return agg_kernel


def _tc_prep(degp, xp):
    def body(degp_ref, x_ref, y_ref, dinv_ref):
        d = degp_ref[0, :, 0:1] + degp_ref[1, :, 0:1] + 1.0
        dinv = lax.rsqrt(d)
        dinv_ref[...] = dinv
        y = x_ref[...] * dinv
        y_ref[0] = y[:, :D1 // 2]
        y_ref[1] = y[:, D1 // 2:]

    return pl.pallas_call(
        body,
        grid=(NPAD // RB,),
        in_specs=[
            pl.BlockSpec((2, RB, 16), lambda i: (0, i, 0)),
            pl.BlockSpec((RB, D1), lambda i: (i, 0)),
        ],
        out_specs=[
            pl.BlockSpec((2, RB, D1 // 2), lambda i: (0, i, 0)),
            pl.BlockSpec((RB, 1), lambda i: (i, 0)),
        ],
        out_shape=[
            jax.ShapeDtypeStruct((2, NPAD, D1 // 2), jnp.float32),
            jax.ShapeDtypeStruct((NPAD, 1), jnp.float32),
        ],
    )(degp, xp)


def _tc_mm(y, p, dinv, W1, b1r, W2p):
    def body(y_ref, p_ref, dinv_ref, w1_ref, b1_ref, w2_ref, y2_ref):
        dinv = dinv_ref[...]
        a = jnp.concatenate(
            [p_ref[0] + y_ref[0], p_ref[1] + y_ref[1]], axis=1) * dinv
        h = lax.dot_general(a, w1_ref[...], (((1,), (1,)), ((), ())),
                            preferred_element_type=jnp.float32)
        h = jnp.maximum(h + b1_ref[...], 0.0)
        m = lax.dot_general(h, w2_ref[...], (((1,), (1,)), ((), ())),
                            preferred_element_type=jnp.float32)
        y2_ref[...] = m * dinv

    return pl.pallas_call(
        body,
        grid=(NPAD // RB,),
        in_specs=[
            pl.BlockSpec((2, RB, D1 // 2), lambda i: (0, i, 0)),
            pl.BlockSpec((2, RB, D1 // 2), lambda i: (0, i, 0)),
            pl.BlockSpec((RB, 1), lambda i: (i, 0)),
            pl.BlockSpec((DH, D1), lambda i: (0, 0)),
            pl.BlockSpec((1, DH), lambda i: (0, 0)),
            pl.BlockSpec((D2P, DH), lambda i: (0, 0)),
        ],
        out_specs=pl.BlockSpec((RB, D2P), lambda i: (i, 0)),
        out_shape=jax.ShapeDtypeStruct((NPAD, D2P), jnp.float32),
    )(y, p, dinv, W1, b1r, W2p)


def _tc_out(q, y2, dinv, b2p):
    def body(q_ref, y2_ref, dinv_ref, b2_ref, o_ref):
        z = (q_ref[0] + q_ref[1] + y2_ref[...]) * dinv_ref[...] + b2_ref[...]
        col = lax.broadcasted_iota(jnp.int32, (RB, D2P), 1)
        zm = jnp.where(col < D2, z, -jnp.inf)
        mx = jnp.max(zm, axis=1, keepdims=True)
        lse = mx + jnp.log(jnp.sum(jnp.exp(zm - mx), axis=1, keepdims=True))
        o_ref[...] = (z - lse)[:, :D2]

    return pl.pallas_call(
        body,
        grid=(NPAD // RB,),
        in_specs=[
            pl.BlockSpec((2, RB, D2P), lambda i: (0, i, 0)),
            pl.BlockSpec((RB, D2P), lambda i: (i, 0)),
            pl.BlockSpec((RB, 1), lambda i: (i, 0)),
            pl.BlockSpec((1, D2P), lambda i: (0, 0)),
        ],
        out_specs=pl.BlockSpec((RB, D2), lambda i: (i, 0)),
        out_shape=jax.ShapeDtypeStruct((N, D2), jnp.float32),
    )(q, y2, dinv, b2p)


_deg_kernel = _make_deg_kernel()
_agg_d1 = _make_agg_split_kernel()
_agg_d2 = _make_agg_kernel(D2P)


def kernel(x, edge_index, W1, b1, W2, b2):
    f32 = jnp.float32
    npad_zone = NPAD - N
    pad_e = EPAD - E
    pad_idx = (N + (jnp.arange(pad_e, dtype=jnp.int32) % npad_zone))
    srcp = jnp.concatenate([edge_index[0], pad_idx]).reshape(NW, NCH, CH)
    dstp = jnp.concatenate([edge_index[1], pad_idx]).reshape(NW, NCH, CH)

    xp = jnp.concatenate([x, jnp.zeros((NPAD - N, D1), f32)], axis=0)
    ones_c = jnp.ones((CH, 16), f32)
    z1 = jnp.zeros((RPS, 16), f32)
    zd1 = jnp.zeros((RPS, D1 // 2), f32)
    zd2 = jnp.zeros((RPS, D2P), f32)
    W2p = jnp.concatenate([W2, jnp.zeros((D2P - D2, DH), f32)], axis=0)
    b1r = b1.reshape(1, DH)
    b2p = jnp.concatenate([b2, jnp.zeros((D2P - D2,), f32)]).reshape(1, D2P)

    padc = (N + (jnp.arange(NBUF * CH, dtype=jnp.int32)
                 % npad_zone)).reshape(NBUF, CH)

    degp = _deg_kernel(dstp, ones_c, z1)
    y, dinv = _tc_prep(degp, xp)
    p = _agg_d1(srcp, dstp, y, zd1, padc)
    y2 = _tc_mm(y, p, dinv, W1, b1r, W2p)
    q = _agg_d2(srcp, dstp, y2, zd2, padc)
    return _tc_out(q, y2, dinv, b2p)
